# Initial kernel scaffold; baseline (speedup 1.0000x reference)
import jax
import jax.numpy as jnp
from jax.experimental import pallas as pl


def kernel(graph_embeddings, target_labels, edge_index, node_graph_ids, W_proj, b_proj, time_table, W_in, b_in, W_h, b_h, W_out, b_out):
    raise NotImplementedError("write your pallas kernel here")


# speedup vs baseline: 3.7850x; 3.7850x over previous
import functools

import jax
import jax.numpy as jnp
import numpy as np
from jax import lax
from jax.experimental import pallas as pl
from jax.experimental.pallas import tpu as pltpu
from jax.experimental.pallas import tpu_sc as plsc

_T = 1000
_N = 50000
_G = 500
_H = 64
_BN = 512
_NB = 98
_N_PAD = _NB * _BN
_GP = 512
_CW = _H + 1
_HALF = _N // 2
_RPS = 1568
_ROWS = 16 * _RPS
_CH = 128
_NCH = 391
_E_PAD = _NCH * _CH * 16


def _schedule():
    betas = np.linspace(1e-4, 0.02, _T, dtype=np.float64)
    alphas_bar = np.cumprod(1.0 - betas)
    return (np.sqrt(alphas_bar).astype(np.float32),
            np.sqrt(1.0 - alphas_bar).astype(np.float32))


def _cond_kernel(tpg_ref, lab_ref, gpe_ref, wp_ref, bp_ref, tt_ref, out_ref):
    tpg = tpg_ref[...]
    iota_t = lax.broadcasted_iota(jnp.int32, (_GP, _T), 1)
    oh_t = (tpg == iota_t).astype(jnp.float32)
    te = jnp.dot(oh_t, tt_ref[...], preferred_element_type=jnp.float32)
    cond = te + jnp.dot(gpe_ref[...], wp_ref[...],
                        preferred_element_type=jnp.float32) + bp_ref[...]
    row = lax.broadcasted_iota(jnp.int32, (_GP, 1), 0)
    valid = (row < _G).astype(jnp.float32)
    out_ref[...] = jnp.concatenate(
        [cond * valid, lab_ref[...] * valid,
         jnp.zeros((_GP, 80 - _CW), jnp.float32)], axis=1)


def _h_kernel(ids_ref, t_ref, noise_ref, cond_ref, sched_ref, wi_ref,
              bi_ref, h_ref, lab_ref):
    ids = ids_ref[...]
    iota_g = lax.broadcasted_iota(jnp.int32, (_BN, _GP), 1)
    oh_g = (ids == iota_g).astype(jnp.float32)
    ce = jnp.dot(oh_g, cond_ref[...], preferred_element_type=jnp.float32)
    labels = ce[:, _H:_H + 1]
    t = t_ref[...]
    iota_t = lax.broadcasted_iota(jnp.int32, (_BN, _T), 1)
    oh_t = (t == iota_t).astype(jnp.float32)
    coef = jnp.dot(oh_t, sched_ref[...], preferred_element_type=jnp.float32)
    noisy = coef[:, 0:1] * labels + coef[:, 1:2] * noise_ref[...]
    h_ref[...] = jax.nn.relu(noisy * wi_ref[...] + bi_ref[...] + ce[:, :_H])
    lab_ref[...] = labels


def _seg_kernel(h_hbm, src_hbm, dst_hbm, outh_hbm, outd_hbm,
                src_v, dst_v, dloc_v, rows_v, zero_v, zed_v, acc, accd, sem):
    c = lax.axis_index("c")
    s = lax.axis_index("s")
    z16 = jnp.zeros((16,), jnp.float32)

    def zrow(r, _):
        for j in range(_H // 16):
            zero_v[r, pl.ds(j * 16, 16)] = z16
        return 0
    lax.fori_loop(0, _CH, zrow, 0)
    for j in range(_CH // 16):
        zed_v[pl.ds(j * 16, 16)] = z16

    zbase = s * _RPS

    def zblk(k, _):
        pltpu.sync_copy(zero_v, acc.at[pl.ds(zbase + k * _CH, _CH)])
        pltpu.sync_copy(zed_v, accd.at[pl.ds(zbase + k * _CH, _CH)])
        return 0
    lax.fori_loop(0, _RPS // _CH, zblk, 0)
    rem = _RPS % _CH
    pltpu.sync_copy(zero_v.at[pl.ds(0, rem)],
                    acc.at[pl.ds(zbase + (_RPS // _CH) * _CH, rem)])
    pltpu.sync_copy(zed_v.at[pl.ds(0, rem)],
                    accd.at[pl.ds(zbase + (_RPS // _CH) * _CH, rem)])
    plsc.subcore_barrier()

    one16 = jnp.ones((16,), jnp.float32)
    for j in range(_CH // 16):
        zed_v[pl.ds(j * 16, 16)] = one16

    ebase = s * (_NCH * _CH)
    base_node = c * _HALF

    def step(j, _):
        off = ebase + j * _CH
        pltpu.sync_copy(src_hbm.at[pl.ds(off, _CH)], src_v)
        pltpu.sync_copy(dst_hbm.at[pl.ds(off, _CH)], dst_v)
        for i in range(_CH // 16):
            d = dst_v[pl.ds(i * 16, 16)] - base_node
            ok = (d >= 0) & (d < _HALF)
            dloc_v[pl.ds(i * 16, 16)] = jnp.where(ok, d, _HALF)
        pltpu.async_copy(h_hbm.at[src_v], rows_v, sem).wait()
        pltpu.sync_copy(rows_v, acc.at[dloc_v], add=True)
        pltpu.sync_copy(zed_v, accd.at[dloc_v], add=True)
        return 0
    lax.fori_loop(0, _NCH, step, 0)
    plsc.subcore_barrier()

    pltpu.sync_copy(acc.at[pl.ds(zbase, _RPS)],
                    outh_hbm.at[c, pl.ds(zbase, _RPS)])
    pltpu.sync_copy(accd.at[pl.ds(zbase, _RPS)],
                    outd_hbm.at[c, pl.ds(zbase, _RPS)])


def _out_kernel(h_ref, lab_ref, ag_ref, deg_ref, wh_ref, bh_ref, wo_ref,
                bo_ref, out_ref):
    pid = pl.program_id(0)
    h = h_ref[...]
    labels = lab_ref[...]
    aggm = ag_ref[...] / jnp.maximum(deg_ref[...], 1.0)
    h2 = jax.nn.relu(jnp.dot(h + aggm, wh_ref[...],
                             preferred_element_type=jnp.float32) + bh_ref[...])
    pred = jnp.dot(h2, wo_ref[...],
                   preferred_element_type=jnp.float32) + bo_ref[...]
    row = pid * _BN + lax.broadcasted_iota(jnp.int32, (_BN, 1), 0)
    mask = (row < _N).astype(jnp.float32)
    ps = jnp.sum((pred - labels) ** 2 * mask).reshape(1, 1)

    @pl.when(pid == 0)
    def _():
        out_ref[...] = ps

    @pl.when(pid != 0)
    def _():
        out_ref[...] = out_ref[...] + ps


def kernel(graph_embeddings, target_labels, edge_index, node_graph_ids,
           W_proj, b_proj, time_table, W_in, b_in, W_h, b_h, W_out, b_out):
    rng = jax.random.key(42)
    k1, k2, k3 = jax.random.split(rng, 3)
    t_nodes = jax.random.randint(k1, (_N,), 0, _T)
    noise = jax.random.normal(k2, (_N, 1), dtype=jnp.float32)
    t_per_graph = jax.random.randint(k3, (_G,), 0, _T)

    sab, somab = _schedule()
    sched = jnp.stack([jnp.asarray(sab), jnp.asarray(somab)], axis=1)

    ids = node_graph_ids.astype(jnp.int32)
    src = edge_index[0].astype(jnp.int32)
    dst = edge_index[1].astype(jnp.int32)
    e = src.shape[0]

    tpg = jnp.pad(t_per_graph.astype(jnp.int32), (0, _GP - _G))[:, None]
    lab = jnp.pad(target_labels.astype(jnp.float32), (0, _GP - _G))[:, None]
    gpe = jnp.pad(graph_embeddings, ((0, _GP - _G), (0, 0)))

    cond_ext = pl.pallas_call(
        _cond_kernel,
        out_shape=jax.ShapeDtypeStruct((_GP, 80), jnp.float32),
    )(tpg, lab, gpe, W_proj, b_proj.reshape(1, _H), time_table)

    ids_p = jnp.pad(ids, (0, _N_PAD - _N), constant_values=_GP - 1)[:, None]
    t_p = jnp.pad(t_nodes.astype(jnp.int32), (0, _N_PAD - _N))[:, None]
    noise_p = jnp.pad(noise, ((0, _N_PAD - _N), (0, 0)))

    h_nodes, lab_nodes = pl.pallas_call(
        _h_kernel,
        grid=(_NB,),
        in_specs=[
            pl.BlockSpec((_BN, 1), lambda i: (i, 0)),
            pl.BlockSpec((_BN, 1), lambda i: (i, 0)),
            pl.BlockSpec((_BN, 1), lambda i: (i, 0)),
            pl.BlockSpec((_GP, 80), lambda i: (0, 0)),
            pl.BlockSpec((_T, 2), lambda i: (0, 0)),
            pl.BlockSpec((1, _H), lambda i: (0, 0)),
            pl.BlockSpec((1, _H), lambda i: (0, 0)),
        ],
        out_specs=[
            pl.BlockSpec((_BN, _H), lambda i: (i, 0)),
            pl.BlockSpec((_BN, 1), lambda i: (i, 0)),
        ],
        out_shape=[
            jax.ShapeDtypeStruct((_N_PAD, _H), jnp.float32),
            jax.ShapeDtypeStruct((_N_PAD, 1), jnp.float32),
        ],
    )(ids_p, t_p, noise_p, cond_ext, sched, W_in, b_in.reshape(1, _H))

    src_p = jnp.pad(src, (0, _E_PAD - e))
    dst_p = jnp.pad(dst, (0, _E_PAD - e), constant_values=_N)

    mesh = plsc.VectorSubcoreMesh(core_axis_name="c", subcore_axis_name="s")
    seg = functools.partial(
        pl.kernel,
        out_type=(
            jax.ShapeDtypeStruct((2, _ROWS, _H), jnp.float32),
            jax.ShapeDtypeStruct((2, _ROWS), jnp.float32),
        ),
        mesh=mesh,
        compiler_params=pltpu.CompilerParams(use_tc_tiling_on_sc=False),
        scratch_types=[
            pltpu.VMEM((_CH,), jnp.int32),
            pltpu.VMEM((_CH,), jnp.int32),
            pltpu.VMEM((_CH,), jnp.int32),
            pltpu.VMEM((_CH, _H), jnp.float32),
            pltpu.VMEM((_CH, _H), jnp.float32),
            pltpu.VMEM((_CH,), jnp.float32),
            pltpu.VMEM_SHARED((_ROWS, _H), jnp.float32),
            pltpu.VMEM_SHARED((_ROWS,), jnp.float32),
            pltpu.SemaphoreType.DMA,
        ],
    )(_seg_kernel)
    aggh2, aggd2 = seg(h_nodes, src_p, dst_p)

    aggh = jnp.concatenate([aggh2[0, :_HALF], aggh2[1, :_HALF]], axis=0)
    aggd = jnp.concatenate([aggd2[0, :_HALF], aggd2[1, :_HALF]], axis=0)
    aggh_p = jnp.pad(aggh, ((0, _N_PAD - _N), (0, 0)))
    aggd_p = jnp.pad(aggd, (0, _N_PAD - _N))[:, None]

    total = pl.pallas_call(
        _out_kernel,
        grid=(_NB,),
        in_specs=[
            pl.BlockSpec((_BN, _H), lambda i: (i, 0)),
            pl.BlockSpec((_BN, 1), lambda i: (i, 0)),
            pl.BlockSpec((_BN, _H), lambda i: (i, 0)),
            pl.BlockSpec((_BN, 1), lambda i: (i, 0)),
            pl.BlockSpec((_H, _H), lambda i: (0, 0)),
            pl.BlockSpec((1, _H), lambda i: (0, 0)),
            pl.BlockSpec((_H, 1), lambda i: (0, 0)),
            pl.BlockSpec((1, 1), lambda i: (0, 0)),
        ],
        out_specs=pl.BlockSpec((1, 1), lambda i: (0, 0)),
        out_shape=jax.ShapeDtypeStruct((1, 1), jnp.float32),
    )(h_nodes, lab_nodes, aggh_p, aggd_p, W_h, b_h.reshape(1, _H),
      W_out, b_out.reshape(1, 1))

    return total[0, 0] / _N
